# baseline (speedup 1.0000x reference)
import jax
import jax.numpy as jnp
from jax.experimental import pallas as pl
from jax.experimental.pallas import tpu as pltpu

_BM = 512


def _fused_body(x_ref, Wb_ref, b_ref, rb1_ref, rW2_ref, rb2g_ref,
                ARb_ref, Bb_ref, E_ref, o_ref):
    nr = ARb_ref.shape[1] - rb1_ref.shape[1]
    x = x_ref[...]
    xb = x.astype(jnp.bfloat16)

    v = jax.lax.dot_general(xb, ARb_ref[...], (((1,), (0,)), ((), ())),
                            preferred_element_type=jnp.float32)
    u = v[:, :nr]
    h = v[:, nr:] + rb1_ref[...]
    h = h * jax.nn.sigmoid(h)
    logits = jax.lax.dot_general(h.astype(jnp.bfloat16), rW2_ref[...],
                                 (((1,), (1,)), ((), ())),
                                 preferred_element_type=jnp.float32)
    logits = logits + rb2g_ref[...]

    idx = jax.lax.broadcasted_iota(jnp.int32, logits.shape, 1)
    m1 = jnp.max(logits, axis=-1, keepdims=True)
    i1 = jnp.min(jnp.where(logits == m1, idx, logits.shape[-1]),
                 axis=-1, keepdims=True)
    masked = jnp.where(idx == i1, -jnp.inf, logits)
    m2 = jnp.max(masked, axis=-1, keepdims=True)
    i2 = jnp.min(jnp.where(masked == m2, idx, logits.shape[-1]),
                 axis=-1, keepdims=True)
    p1 = jax.nn.sigmoid(m1 - m2)
    coeff = jnp.where(idx == i1, p1, jnp.where(idx == i2, 1.0 - p1, 0.0))

    C = jnp.dot(coeff, E_ref[...], preferred_element_type=jnp.float32)
    uw = (u * C).astype(jnp.bfloat16)
    delta = jnp.dot(uw, Bb_ref[...], preferred_element_type=jnp.float32)

    base = jax.lax.dot_general(xb, Wb_ref[...], (((1,), (0,)), ((), ())),
                               preferred_element_type=jnp.float32)

    o_ref[...] = base + delta + b_ref[...]


def kernel(x, W, b, rW1, rb1, rW2, rb2, gates, A, B):
    n_tokens, d_in = x.shape
    d_out = W.shape[0]
    num_experts, rank = A.shape[0], A.shape[1]
    r_hid = rW1.shape[0]

    Wb = W.T.astype(jnp.bfloat16)
    ARb = jnp.concatenate(
        [A.reshape(num_experts * rank, d_in), rW1],
        axis=0).T.astype(jnp.bfloat16)
    Bb = jnp.transpose(B, (0, 2, 1)).reshape(
        num_experts * rank, d_out).astype(jnp.bfloat16)
    rW2e = rW2[:num_experts].astype(jnp.bfloat16)
    rb2g = (rb2[:num_experts] + gates).reshape(1, num_experts)
    E = jnp.kron(jnp.eye(num_experts, dtype=jnp.float32),
                 jnp.ones((1, rank), dtype=jnp.float32))

    bm = _BM
    grid = (n_tokens // bm,)

    full = lambda shape: pl.BlockSpec(shape, lambda i: (0,) * len(shape))
    out = pl.pallas_call(
        _fused_body,
        grid=grid,
        in_specs=[
            pl.BlockSpec((bm, d_in), lambda i: (i, 0)),
            full((d_in, d_out)),
            full((1, d_out)),
            full((1, r_hid)),
            full((num_experts, r_hid)),
            full((1, num_experts)),
            full((d_in, num_experts * rank + r_hid)),
            full((num_experts * rank, d_out)),
            full((num_experts, num_experts * rank)),
        ],
        out_specs=pl.BlockSpec((bm, d_out), lambda i: (i, 0)),
        out_shape=jax.ShapeDtypeStruct((n_tokens, d_out), jnp.float32),
        compiler_params=pltpu.CompilerParams(
            dimension_semantics=("arbitrary",)),
    )(x, Wb, b.reshape(1, d_out), rb1.reshape(1, r_hid),
      rW2e, rb2g, ARb, Bb, E)
    return out


# speedup vs baseline: 1.0295x; 1.0295x over previous
import jax
import jax.numpy as jnp
from jax.experimental import pallas as pl
from jax.experimental.pallas import tpu as pltpu

_BM = 512


def _fused_body(x_ref, Wb_ref, b_ref, rb1_ref, rW2_ref, rb2g_ref,
                ARb_ref, Bb_ref, E_ref, o_ref):
    nr = ARb_ref.shape[0] - rb1_ref.shape[1]
    x = x_ref[...]
    xb = x.astype(jnp.bfloat16)

    v = jax.lax.dot_general(xb, ARb_ref[...], (((1,), (1,)), ((), ())),
                            preferred_element_type=jnp.float32)
    u = v[:, :nr]
    h = v[:, nr:] + rb1_ref[...]
    h = h * jax.nn.sigmoid(h)
    logits = jax.lax.dot_general(h.astype(jnp.bfloat16), rW2_ref[...],
                                 (((1,), (1,)), ((), ())),
                                 preferred_element_type=jnp.float32)
    logits = logits + rb2g_ref[...]

    idx = jax.lax.broadcasted_iota(jnp.int32, logits.shape, 1)
    m1 = jnp.max(logits, axis=-1, keepdims=True)
    i1 = jnp.min(jnp.where(logits == m1, idx, logits.shape[-1]),
                 axis=-1, keepdims=True)
    masked = jnp.where(idx == i1, -jnp.inf, logits)
    m2 = jnp.max(masked, axis=-1, keepdims=True)
    i2 = jnp.min(jnp.where(masked == m2, idx, logits.shape[-1]),
                 axis=-1, keepdims=True)
    p1 = jax.nn.sigmoid(m1 - m2)
    coeff = jnp.where(idx == i1, p1, jnp.where(idx == i2, 1.0 - p1, 0.0))

    C = jnp.dot(coeff, E_ref[...], preferred_element_type=jnp.float32)
    uw = (u * C).astype(jnp.bfloat16)
    delta = jnp.dot(uw, Bb_ref[...], preferred_element_type=jnp.float32)

    base = jax.lax.dot_general(xb, Wb_ref[...], (((1,), (1,)), ((), ())),
                               preferred_element_type=jnp.float32)

    o_ref[...] = base + delta + b_ref[...]


def kernel(x, W, b, rW1, rb1, rW2, rb2, gates, A, B):
    n_tokens, d_in = x.shape
    d_out = W.shape[0]
    num_experts, rank = A.shape[0], A.shape[1]
    r_hid = rW1.shape[0]

    Wb = W.astype(jnp.bfloat16)
    ARb = jnp.concatenate(
        [A.reshape(num_experts * rank, d_in), rW1],
        axis=0).astype(jnp.bfloat16)
    Bb = jnp.transpose(B, (0, 2, 1)).reshape(
        num_experts * rank, d_out).astype(jnp.bfloat16)
    rW2e = rW2[:num_experts].astype(jnp.bfloat16)
    rb2g = (rb2[:num_experts] + gates).reshape(1, num_experts)
    E = jnp.kron(jnp.eye(num_experts, dtype=jnp.float32),
                 jnp.ones((1, rank), dtype=jnp.float32))

    bm = _BM
    grid = (n_tokens // bm,)

    full = lambda shape: pl.BlockSpec(shape, lambda i: (0,) * len(shape))
    out = pl.pallas_call(
        _fused_body,
        grid=grid,
        in_specs=[
            pl.BlockSpec((bm, d_in), lambda i: (i, 0)),
            full((d_out, d_in)),
            full((1, d_out)),
            full((1, r_hid)),
            full((num_experts, r_hid)),
            full((1, num_experts)),
            full((num_experts * rank + r_hid, d_in)),
            full((num_experts * rank, d_out)),
            full((num_experts, num_experts * rank)),
        ],
        out_specs=pl.BlockSpec((bm, d_out), lambda i: (i, 0)),
        out_shape=jax.ShapeDtypeStruct((n_tokens, d_out), jnp.float32),
        compiler_params=pltpu.CompilerParams(
            dimension_semantics=("parallel",)),
    )(x, Wb, b.reshape(1, d_out), rb1.reshape(1, r_hid),
      rW2e, rb2g, ARb, Bb, E)
    return out


# speedup vs baseline: 1.0617x; 1.0312x over previous
import jax
import jax.numpy as jnp
from jax.experimental import pallas as pl
from jax.experimental.pallas import tpu as pltpu

_BM = 512


def _fused_body(x_ref, Wb_ref, b_ref, rb1_ref, rW2_ref, rb2g_ref,
                ARb_ref, Bb_ref, E_ref, o_ref):
    nr = ARb_ref.shape[0] - rb1_ref.shape[1]
    x = x_ref[...]
    xb = x.astype(jnp.bfloat16)

    v = jax.lax.dot_general(xb, ARb_ref[...], (((1,), (1,)), ((), ())),
                            preferred_element_type=jnp.float32)
    u = v[:, :nr]
    h = v[:, nr:] + rb1_ref[...]
    h = h * jax.nn.sigmoid(h)
    logits = jax.lax.dot_general(h.astype(jnp.bfloat16), rW2_ref[...],
                                 (((1,), (1,)), ((), ())),
                                 preferred_element_type=jnp.float32)
    logits = logits + rb2g_ref[...]

    m1 = jnp.max(logits, axis=-1, keepdims=True)
    top1 = logits == m1
    masked = jnp.where(top1, -jnp.inf, logits)
    m2 = jnp.max(masked, axis=-1, keepdims=True)
    p1 = jax.nn.sigmoid(m1 - m2)
    coeff = jnp.where(top1, p1, jnp.where(masked == m2, 1.0 - p1, 0.0))

    C = jnp.dot(coeff.astype(jnp.bfloat16), E_ref[...],
                preferred_element_type=jnp.float32)
    uw = (u * C).astype(jnp.bfloat16)
    delta = jnp.dot(uw, Bb_ref[...], preferred_element_type=jnp.float32)

    base = jax.lax.dot_general(xb, Wb_ref[...], (((1,), (1,)), ((), ())),
                               preferred_element_type=jnp.float32)

    o_ref[...] = base + delta + b_ref[...]


def kernel(x, W, b, rW1, rb1, rW2, rb2, gates, A, B):
    n_tokens, d_in = x.shape
    d_out = W.shape[0]
    num_experts, rank = A.shape[0], A.shape[1]
    r_hid = rW1.shape[0]

    Wb = W.astype(jnp.bfloat16)
    ARb = jnp.concatenate(
        [A.reshape(num_experts * rank, d_in), rW1],
        axis=0).astype(jnp.bfloat16)
    Bb = jnp.transpose(B, (0, 2, 1)).reshape(
        num_experts * rank, d_out).astype(jnp.bfloat16)
    rW2e = rW2[:num_experts].astype(jnp.bfloat16)
    rb2g = (rb2[:num_experts] + gates).reshape(1, num_experts)
    E = jnp.kron(jnp.eye(num_experts, dtype=jnp.float32),
                 jnp.ones((1, rank), dtype=jnp.float32)).astype(jnp.bfloat16)

    bm = _BM
    grid = (n_tokens // bm,)

    full = lambda shape: pl.BlockSpec(shape, lambda i: (0,) * len(shape))
    out = pl.pallas_call(
        _fused_body,
        grid=grid,
        in_specs=[
            pl.BlockSpec((bm, d_in), lambda i: (i, 0)),
            full((d_out, d_in)),
            full((1, d_out)),
            full((1, r_hid)),
            full((num_experts, r_hid)),
            full((1, num_experts)),
            full((num_experts * rank + r_hid, d_in)),
            full((num_experts * rank, d_out)),
            full((num_experts, num_experts * rank)),
        ],
        out_specs=pl.BlockSpec((bm, d_out), lambda i: (i, 0)),
        out_shape=jax.ShapeDtypeStruct((n_tokens, d_out), jnp.float32),
        compiler_params=pltpu.CompilerParams(
            dimension_semantics=("parallel",)),
    )(x, Wb, b.reshape(1, d_out), rb1.reshape(1, r_hid),
      rW2e, rb2g, ARb, Bb, E)
    return out


# speedup vs baseline: 1.2220x; 1.1510x over previous
import jax
import jax.numpy as jnp
from jax.experimental import pallas as pl
from jax.experimental.pallas import tpu as pltpu

_BM = 512


def _fused_body(x_ref, W_ref, b_ref, rb1_ref, rW2_ref, rb2g_ref,
                ARb_ref, Bb_ref, E_ref, o_ref, xb_s, uw_s, Wb_s):
    nr = Bb_ref.shape[0]

    @pl.when(pl.program_id(0) == 0)
    def _():
        Wb_s[...] = W_ref[...].astype(jnp.bfloat16)

    base = jax.lax.dot_general(xb_s[...], Wb_s[...], (((1,), (1,)), ((), ())),
                               preferred_element_type=jnp.float32)
    delta = jnp.dot(uw_s[...], Bb_ref[...], preferred_element_type=jnp.float32)
    o_ref[...] = base + delta + b_ref[...]

    xb = x_ref[...].astype(jnp.bfloat16)
    v = jax.lax.dot_general(xb, ARb_ref[...], (((1,), (1,)), ((), ())),
                            preferred_element_type=jnp.float32)
    u = v[:, :nr]
    h = v[:, nr:] + rb1_ref[...]
    h = h * jax.nn.sigmoid(h)
    logits = jax.lax.dot_general(h.astype(jnp.bfloat16), rW2_ref[...],
                                 (((1,), (1,)), ((), ())),
                                 preferred_element_type=jnp.float32)
    logits = logits + rb2g_ref[...]

    m1 = jnp.max(logits, axis=-1, keepdims=True)
    top1 = logits == m1
    masked = jnp.where(top1, -jnp.inf, logits)
    m2 = jnp.max(masked, axis=-1, keepdims=True)
    p1 = jax.nn.sigmoid(m1 - m2)
    coeff = jnp.where(top1, p1, jnp.where(masked == m2, 1.0 - p1, 0.0))

    C = jnp.dot(coeff.astype(jnp.bfloat16), E_ref[...],
                preferred_element_type=jnp.float32)
    uw_s[...] = (u * C).astype(jnp.bfloat16)
    xb_s[...] = xb


def kernel(x, W, b, rW1, rb1, rW2, rb2, gates, A, B):
    n_tokens, d_in = x.shape
    d_out = W.shape[0]
    num_experts, rank = A.shape[0], A.shape[1]
    r_hid = rW1.shape[0]

    ARb = jnp.concatenate(
        [A.reshape(num_experts * rank, d_in), rW1],
        axis=0).astype(jnp.bfloat16)
    Bb = jnp.transpose(B, (0, 2, 1)).reshape(
        num_experts * rank, d_out).astype(jnp.bfloat16)
    rW2e = rW2[:num_experts].astype(jnp.bfloat16)
    rb2g = (rb2[:num_experts] + gates).reshape(1, num_experts)
    E = jnp.kron(jnp.eye(num_experts, dtype=jnp.float32),
                 jnp.ones((1, rank), dtype=jnp.float32)).astype(jnp.bfloat16)

    bm = _BM
    nm = n_tokens // bm
    grid = (nm + 1,)

    full = lambda shape: pl.BlockSpec(shape, lambda i: (0,) * len(shape))
    out = pl.pallas_call(
        _fused_body,
        grid=grid,
        in_specs=[
            pl.BlockSpec((bm, d_in), lambda i: (jnp.minimum(i, nm - 1), 0)),
            full((d_out, d_in)),
            full((1, d_out)),
            full((1, r_hid)),
            full((num_experts, r_hid)),
            full((1, num_experts)),
            full((num_experts * rank + r_hid, d_in)),
            full((num_experts * rank, d_out)),
            full((num_experts, num_experts * rank)),
        ],
        out_specs=pl.BlockSpec((bm, d_out),
                               lambda i: (jnp.maximum(i - 1, 0), 0)),
        out_shape=jax.ShapeDtypeStruct((n_tokens, d_out), jnp.float32),
        scratch_shapes=[
            pltpu.VMEM((bm, d_in), jnp.bfloat16),
            pltpu.VMEM((bm, num_experts * rank), jnp.bfloat16),
            pltpu.VMEM((d_out, d_in), jnp.bfloat16),
        ],
        compiler_params=pltpu.CompilerParams(
            dimension_semantics=("arbitrary",)),
    )(x, W, b.reshape(1, d_out), rb1.reshape(1, r_hid),
      rW2e, rb2g, ARb, Bb, E)
    return out
